# baseline (speedup 1.0000x reference)
import jax
import jax.numpy as jnp
from jax import lax
from jax.experimental import pallas as pl
from jax.experimental.pallas import tpu as pltpu
from jax.experimental.pallas import tpu_sc as plsc

D = 16
B = 16384
N = 100000
NC, NS, L = 2, 16, 16
NW = NC * NS
PER_W = B // NW
CH = 128
NCHUNK = PER_W // CH
NG = CH // L

AC = 12288


def _pack_body(ta_ref, tb_ref, a_ref, b_ref):
    a_ref[...] = ta_ref[...].T
    b_ref[...] = tb_ref[...].T


_pack = pl.pallas_call(
    _pack_body,
    grid=(pl.cdiv(N, AC),),
    in_specs=[pl.BlockSpec((128, AC), lambda k: (0, k)),
              pl.BlockSpec((128, AC), lambda k: (1, k))],
    out_specs=[pl.BlockSpec((AC, 128), lambda k: (k, 0)),
               pl.BlockSpec((AC, 128), lambda k: (k, 0))],
    out_shape=[jax.ShapeDtypeStruct((N, 128), jnp.float32),
               jax.ShapeDtypeStruct((N, 128), jnp.float32)],
)


def _sc_body(ann_hbm, mu_hbm, covA_hbm, covB_hbm, eps_hbm, z_hbm,
             idx_v, covA_v, covB_v, mu_v, eps_v, out_v, sem):
    wid = lax.axis_index("s") * NC + lax.axis_index("c")
    base = wid * PER_W
    lane = lax.iota(jnp.int32, L)

    for c in range(NCHUNK):
        off = base + c * CH
        pltpu.sync_copy(ann_hbm.at[pl.ds(off, CH)], idx_v)
        cpA = pltpu.async_copy(covA_hbm.at[idx_v], covA_v, sem)
        cpB = pltpu.async_copy(covB_hbm.at[idx_v], covB_v, sem)
        cpM = pltpu.async_copy(mu_hbm.at[idx_v], mu_v, sem)
        pltpu.sync_copy(eps_hbm.at[pl.ds(off, CH)], eps_v)
        cpA.wait()
        cpB.wait()
        cpM.wait()

        def group(g, carry):
            s_idx = g * L + lane
            e = [plsc.load_gather(eps_v, [s_idx, jnp.full((L,), j, jnp.int32)])
                 for j in range(D)]
            for i in range(D):
                z0 = plsc.load_gather(
                    mu_v, [s_idx, jnp.full((L,), i, jnp.int32)])
                z1 = None
                for j in range(i + 1):
                    col = i * D + j
                    src = covA_v if col < 128 else covB_v
                    cij = plsc.load_gather(
                        src, [s_idx, jnp.full((L,), col % 128, jnp.int32)])
                    if j % 2 == 0:
                        z0 = z0 + cij * e[j]
                    else:
                        z1 = cij * e[j] if z1 is None else z1 + cij * e[j]
                z = z0 if z1 is None else z0 + z1
                plsc.store_scatter(out_v, [s_idx, jnp.full((L,), i, jnp.int32)], z)
            return carry

        lax.fori_loop(0, NG, group, None)
        pltpu.sync_copy(out_v, z_hbm.at[pl.ds(off, CH)])


def _make_sc_kernel(interpret=False):
    return pl.kernel(
        _sc_body,
        out_type=jax.ShapeDtypeStruct((B, D), jnp.float32),
        mesh=plsc.VectorSubcoreMesh(core_axis_name="c", subcore_axis_name="s",
                                    num_cores=NC, num_subcores=NS),
        scratch_types=[
            pltpu.VMEM((CH,), jnp.int32),
            pltpu.VMEM((CH, 128), jnp.float32),
            pltpu.VMEM((CH, 128), jnp.float32),
            pltpu.VMEM((CH, D), jnp.float32),
            pltpu.VMEM((CH, D), jnp.float32),
            pltpu.VMEM((CH, D), jnp.float32),
            pltpu.SemaphoreType.DMA,
        ],
        compiler_params=pltpu.CompilerParams(needs_layout_passes=False,
                                             use_tc_tiling_on_sc=False),
        interpret=interpret,
    )


_EPS_CACHE = []


def _get_eps():
    if not _EPS_CACHE:
        try:
            with jax.ensure_compile_time_eval():
                _EPS_CACHE.append(jax.random.normal(
                    jax.random.fold_in(jax.random.key(1), 7), (B, D),
                    jnp.float32))
        except Exception:
            return jax.random.normal(
                jax.random.fold_in(jax.random.key(1), 7), (B, D), jnp.float32)
    return _EPS_CACHE[0]


def kernel(annotator, posterior_mu, posterior_cov):
    covT = jnp.transpose(posterior_cov, (1, 2, 0)).reshape(D * D, N)
    covA, covB = _pack(covT, covT)
    return _make_sc_kernel()(annotator.astype(jnp.int32), posterior_mu,
                             covA, covB, _get_eps())


# speedup vs baseline: 1.0046x; 1.0046x over previous
import jax
import jax.numpy as jnp
from jax import lax
from jax.experimental import pallas as pl
from jax.experimental.pallas import tpu as pltpu
from jax.experimental.pallas import tpu_sc as plsc

D = 16
B = 16384
N = 100000
NC, NS, L = 2, 16, 16
NW = NC * NS
PER_W = B // NW
CH = 128
NCHUNK = PER_W // CH
NG = CH // L

AC = 8192


def _pack_body(ta_ref, tb_ref, a_ref, b_ref):
    a_ref[...] = ta_ref[...].T
    b_ref[...] = tb_ref[...].T


_pack = pl.pallas_call(
    _pack_body,
    grid=(pl.cdiv(N, AC),),
    in_specs=[pl.BlockSpec((128, AC), lambda k: (0, k)),
              pl.BlockSpec((128, AC), lambda k: (1, k))],
    out_specs=[pl.BlockSpec((AC, 128), lambda k: (k, 0)),
               pl.BlockSpec((AC, 128), lambda k: (k, 0))],
    out_shape=[jax.ShapeDtypeStruct((N, 128), jnp.float32),
               jax.ShapeDtypeStruct((N, 128), jnp.float32)],
)


def _sc_body(ann_hbm, mu_hbm, covA_hbm, covB_hbm, eps_hbm, z_hbm,
             idx_v, covA_v, covB_v, mu_v, eps_v, out_v, sem):
    wid = lax.axis_index("s") * NC + lax.axis_index("c")
    base = wid * PER_W
    lane = lax.iota(jnp.int32, L)

    for c in range(NCHUNK):
        off = base + c * CH
        pltpu.sync_copy(ann_hbm.at[pl.ds(off, CH)], idx_v)
        cpA = pltpu.async_copy(covA_hbm.at[idx_v], covA_v, sem)
        cpB = pltpu.async_copy(covB_hbm.at[idx_v], covB_v, sem)
        cpM = pltpu.async_copy(mu_hbm.at[idx_v], mu_v, sem)
        pltpu.sync_copy(eps_hbm.at[pl.ds(off, CH)], eps_v)
        cpA.wait()
        cpB.wait()
        cpM.wait()

        def group(g, carry):
            s_idx = g * L + lane
            e = [plsc.load_gather(eps_v, [s_idx, jnp.full((L,), j, jnp.int32)])
                 for j in range(D)]
            for i in range(D):
                z0 = plsc.load_gather(
                    mu_v, [s_idx, jnp.full((L,), i, jnp.int32)])
                z1 = None
                for j in range(i + 1):
                    col = i * D + j
                    src = covA_v if col < 128 else covB_v
                    cij = plsc.load_gather(
                        src, [s_idx, jnp.full((L,), col % 128, jnp.int32)])
                    if j % 2 == 0:
                        z0 = z0 + cij * e[j]
                    else:
                        z1 = cij * e[j] if z1 is None else z1 + cij * e[j]
                z = z0 if z1 is None else z0 + z1
                plsc.store_scatter(out_v, [s_idx, jnp.full((L,), i, jnp.int32)], z)
            return carry

        lax.fori_loop(0, NG, group, None)
        pltpu.sync_copy(out_v, z_hbm.at[pl.ds(off, CH)])


def _make_sc_kernel(interpret=False):
    return pl.kernel(
        _sc_body,
        out_type=jax.ShapeDtypeStruct((B, D), jnp.float32),
        mesh=plsc.VectorSubcoreMesh(core_axis_name="c", subcore_axis_name="s",
                                    num_cores=NC, num_subcores=NS),
        scratch_types=[
            pltpu.VMEM((CH,), jnp.int32),
            pltpu.VMEM((CH, 128), jnp.float32),
            pltpu.VMEM((CH, 128), jnp.float32),
            pltpu.VMEM((CH, D), jnp.float32),
            pltpu.VMEM((CH, D), jnp.float32),
            pltpu.VMEM((CH, D), jnp.float32),
            pltpu.SemaphoreType.DMA,
        ],
        compiler_params=pltpu.CompilerParams(needs_layout_passes=False,
                                             use_tc_tiling_on_sc=False),
        interpret=interpret,
    )


_EPS_CACHE = []


def _get_eps():
    if not _EPS_CACHE:
        try:
            with jax.ensure_compile_time_eval():
                _EPS_CACHE.append(jax.random.normal(
                    jax.random.fold_in(jax.random.key(1), 7), (B, D),
                    jnp.float32))
        except Exception:
            return jax.random.normal(
                jax.random.fold_in(jax.random.key(1), 7), (B, D), jnp.float32)
    return _EPS_CACHE[0]


def kernel(annotator, posterior_mu, posterior_cov):
    covT = jnp.transpose(posterior_cov, (1, 2, 0)).reshape(D * D, N)
    covA, covB = _pack(covT, covT)
    return _make_sc_kernel()(annotator.astype(jnp.int32), posterior_mu,
                             covA, covB, _get_eps())
